# baseline (speedup 1.0000x reference)
import functools

import numpy as np
import jax
import jax.numpy as jnp
from jax import lax
from jax.experimental import pallas as pl
from jax.experimental.pallas import tpu as pltpu
from jax.experimental.pallas import tpu_sc as plsc

_f32 = jnp.float32

N = 10000
NPAD = 10240
E = 320000
NW = 32
K = 56
NB = 18
NBLK = 10
B0 = 8
B1 = 12
NCH = NB * NBLK
EPW = K * NCH
EPAD = NW * EPW
NTILES = 16
RPT = NPAD // NTILES
GARBAGE = N

_IH16 = np.zeros((64, 16), np.float32)
for _l in range(64):
    _IH16[_l, _l % 8] = 1.0
    _IH16[_l, 8 + _l % 8] = 1.0


def _leaky_exp(t):
    return jnp.exp(jnp.where(t > 0, t, 0.2 * t))


def _ph_a_body(x_ref, w1p, ms, md, st_o, ad_o, init_o):
    xpc = jnp.dot(x_ref[...], w1p[...], preferred_element_type=_f32)
    as16 = jnp.dot(xpc, ms[...], preferred_element_type=_f32)
    ad16 = jnp.dot(xpc, md[...], preferred_element_type=_f32)
    ex = _leaky_exp(as16 + ad16)
    st_o[...] = jnp.concatenate([xpc, as16], axis=1)
    ad_o[...] = ad16
    init_o[...] = jnp.concatenate(
        [xpc * jnp.concatenate([ex[:, 0:8]] * 8, axis=1), ex], axis=1)


_RB = 2048


def _ph_a(xpad, w1p, ms16, md16):
    g = NPAD // _RB
    full = lambda s: pl.BlockSpec(s, lambda i: (0, 0))
    return pl.pallas_call(
        _ph_a_body,
        grid=(g,),
        in_specs=[
            pl.BlockSpec((_RB, 128), lambda i: (i, 0)),
            full((128, 64)), full((64, 16)), full((64, 16)),
        ],
        out_specs=[
            pl.BlockSpec((_RB, 80), lambda i: (i, 0)),
            pl.BlockSpec((_RB, 16), lambda i: (i, 0)),
            pl.BlockSpec((_RB, 80), lambda i: (i, 0)),
        ],
        out_shape=[
            jax.ShapeDtypeStruct((NPAD, 80), _f32),
            jax.ShapeDtypeStruct((NPAD, 16), _f32),
            jax.ShapeDtypeStruct((NPAD, 80), _f32),
        ],
    )(xpad, w1p, ms16, md16)


def _ph_c_body(s0, s1, b1p, w2p, a2s, a2d, st_o, ad_o, init_o):
    sm = s0[...] + s1[...]
    num = sm[:, 0:64]
    den = jnp.concatenate([sm[:, 64:72]] * 8, axis=1)
    o1 = num / (den + 1e-16) + b1p[...]
    h1 = jnp.where(o1 > 0, o1, jnp.exp(jnp.minimum(o1, 0.0)) - 1.0)
    xp2 = jnp.dot(h1, w2p[...], preferred_element_type=_f32)
    as16 = jnp.dot(xp2, a2s[...], preferred_element_type=_f32)
    ad16 = jnp.dot(xp2, a2d[...], preferred_element_type=_f32)
    ex2 = _leaky_exp(as16 + ad16)
    st_o[...] = jnp.concatenate([xp2, as16], axis=1)
    ad_o[...] = ad16
    init_o[...] = jnp.concatenate(
        [xp2 * jnp.concatenate([ex2] * 8, axis=1), ex2], axis=1)


def _ph_c(acc1, b1p2d, w2p, a2s16, a2d16):
    g = NPAD // _RB
    full = lambda s: pl.BlockSpec(s, lambda i: (0, 0))
    return pl.pallas_call(
        _ph_c_body,
        grid=(g,),
        in_specs=[
            pl.BlockSpec((_RB, 80), lambda i: (i, 0)),
            pl.BlockSpec((_RB, 80), lambda i: (i + NPAD // _RB, 0)),
            full((1, 64)),
            full((64, 128)), full((128, 16)), full((128, 16)),
        ],
        out_specs=[
            pl.BlockSpec((_RB, 144), lambda i: (i, 0)),
            pl.BlockSpec((_RB, 16), lambda i: (i, 0)),
            pl.BlockSpec((_RB, 144), lambda i: (i, 0)),
        ],
        out_shape=[
            jax.ShapeDtypeStruct((NPAD, 144), _f32),
            jax.ShapeDtypeStruct((NPAD, 16), _f32),
            jax.ShapeDtypeStruct((NPAD, 144), _f32),
        ],
    )(acc1, acc1, b1p2d, w2p, a2s16, a2d16)


def _ph_e_body(s0, s1, b2r, w8, lb8, y_o):
    sm = s0[...] + s1[...]
    num = sm[:, 0:128]
    den = jnp.concatenate([sm[:, 128:144]] * 8, axis=1)
    o2 = num / (den + 1e-16) + b2r[...]
    y_o[...] = jnp.dot(o2, w8[...], preferred_element_type=_f32) + lb8[...]


def _ph_e(acc2, b2r, w8, lb8):
    g = NPAD // _RB
    full = lambda s: pl.BlockSpec(s, lambda i: (0, 0))
    return pl.pallas_call(
        _ph_e_body,
        grid=(g,),
        in_specs=[
            pl.BlockSpec((_RB, 144), lambda i: (i, 0)),
            pl.BlockSpec((_RB, 144), lambda i: (i + NPAD // _RB, 0)),
            full((1, 128)), full((128, 8)), full((1, 8)),
        ],
        out_specs=pl.BlockSpec((_RB, 8), lambda i: (i, 0)),
        out_shape=jax.ShapeDtypeStruct((NPAD, 8), _f32),
    )(acc2, acc2, b2r, w8, lb8)


def _make_sc_edge(D, DO):
    mesh = plsc.VectorSubcoreMesh(core_axis_name="c", subcore_axis_name="s")
    vm = pltpu.VMEM
    sem = pltpu.SemaphoreType.DMA

    @functools.partial(
        pl.kernel,
        mesh=mesh,
        out_type=jax.ShapeDtypeStruct((2 * NPAD, DO), _f32),
        compiler_params=pltpu.CompilerParams(use_tc_tiling_on_sc=False),
        scratch_types=[
            [vm((NB, K), jnp.int32)] * 2, [vm((NB, K), jnp.int32)] * 2,
            [vm((K, 16), _f32)] * 2, [vm((K, DO), _f32)] * 2,
            [vm((K, DO), _f32)] * 2,
            pltpu.VMEM_SHARED((NPAD, DO), _f32),
            [sem] * 2, [sem] * 2, [sem] * 2, [sem] * 2, [sem] * 2,
        ],
    )
    def sc(st_h, ad_h, src4_h, dst4_h, init_h, zro_h, out_h,
           ibs, ibd, bv, fv, ov, snum, sis, sid, ga, gb, gs):
        c = lax.axis_index("c")
        s = lax.axis_index("s")
        row0 = s * RPT

        @pl.when(c == 0)
        def _():
            pltpu.sync_copy(init_h.at[pl.ds(row0, RPT)],
                            snum.at[pl.ds(row0, RPT)])

        @pl.when(c == 1)
        def _():
            pltpu.sync_copy(zro_h.at[pl.ds(row0, RPT)],
                            snum.at[pl.ds(row0, RPT)])

        plsc.subcore_barrier()
        bbase = c * (NTILES * B0) + s * (B0 + (B1 - B0) * c)
        nblk = B0 + (B1 - B0) * c

        def idx_start(b, q):
            pltpu.make_async_copy(src4_h.at[bbase + b], ibs[q], sis[q]).start()
            pltpu.make_async_copy(dst4_h.at[bbase + b], ibd[q], sid[q]).start()

        def idx_wait(q):
            pltpu.make_async_copy(src4_h.at[bbase], ibs[q], sis[q]).wait()
            pltpu.make_async_copy(dst4_h.at[bbase], ibd[q], sid[q]).wait()

        def g_start(q, j, p):
            pltpu.make_async_copy(st_h.at[ibs[q].at[j]], fv[p], ga[p]).start()
            pltpu.make_async_copy(ad_h.at[ibd[q].at[j]], bv[p], gb[p]).start()

        def g_wait(q, p):
            pltpu.make_async_copy(st_h.at[ibs[q].at[0]], fv[p], ga[p]).wait()
            pltpu.make_async_copy(ad_h.at[ibd[q].at[0]], bv[p], gb[p]).wait()

        def sc_start(q, j, p):
            pltpu.make_async_copy(ov[p], snum.at[ibd[q].at[j]],
                                  gs[p]).start(add=True)

        def sc_drain(q, p):
            pltpu.make_async_copy(ov[p], snum.at[ibd[q].at[0]], gs[p]).wait()

        idx_start(0, 0)

        def blockpair(b2, cr):
            for q in (0, 1):
                b = b2 * 2 + q
                idx_wait(q)
                g_start(q, 0, 0)
                g_start(q, 1, 1)

                def pairbody(pj, cr2, q=q, b=b):
                    for p in (0, 1):
                        j = pj * 2 + p
                        ch = b * NB + j
                        g_wait(q, p)

                        @pl.when(ch >= 2)
                        def _(q=q, p=p):
                            sc_drain(q, p)

                        def ebody(i2, cr3, q=q, p=p):
                            ee = (i2 * 2, i2 * 2 + 1)
                            exs = [_leaky_exp(
                                fv[p][e, D:D + 16] + bv[p][e, :]) for e in ee]
                            for e, ex in zip(ee, exs):
                                fs = [fv[p][e, k * 16:(k + 1) * 16]
                                      for k in range(D // 16)]
                                outs = [f * ex for f in fs]
                                for k in range(D // 16):
                                    ov[p][e, k * 16:(k + 1) * 16] = outs[k]
                                ov[p][e, D:D + 16] = ex
                            return cr3

                        lax.fori_loop(0, K // 2, ebody, 0, unroll=2)
                        sc_start(q, j, p)

                        @pl.when(j + 2 < NB)
                        def _(q=q, j=j, p=p):
                            g_start(q, j + 2, p)

                    @pl.when((pj == 0) & (b + 1 < nblk))
                    def _(q=q, b=b):
                        idx_start(b + 1, 1 - q)
                    return cr2

                lax.fori_loop(0, NB // 2, pairbody, 0)
            return cr

        lax.fori_loop(0, nblk // 2, blockpair, 0)
        sc_drain(1, 0)
        sc_drain(1, 1)
        plsc.subcore_barrier()
        pltpu.sync_copy(snum.at[pl.ds(row0, RPT)],
                        out_h.at[pl.ds(c * NPAD + row0, RPT)])

    return sc


_sc_cache = {}


def _sc_edge(D, DO):
    if (D, DO) not in _sc_cache:
        _sc_cache[(D, DO)] = _make_sc_edge(D, DO)
    return _sc_cache[(D, DO)]


def kernel(x, edge_index, W1, a1_src, a1_dst, b1, W2, a2_src, a2_dst, b2,
           lin_W, lin_b):
    w1p = W1.reshape(128, 8, 8).transpose(0, 2, 1).reshape(128, 64)
    ih16 = jnp.asarray(_IH16)
    ms16 = a1_src.T.reshape(64, 1) * ih16
    md16 = a1_dst.T.reshape(64, 1) * ih16
    b1p2d = b1.reshape(8, 8).T.reshape(1, 64)
    w2p = W2.reshape(8, 8, 128).transpose(1, 0, 2).reshape(64, 128)
    a2s16 = jnp.tile(a2_src.reshape(128, 1), (1, 16))
    a2d16 = jnp.tile(a2_dst.reshape(128, 1), (1, 16))
    b2r = b2.reshape(1, 128)
    w8 = jnp.tile(lin_W.reshape(128, 1), (1, 8))
    lb8 = jnp.broadcast_to(lin_b.reshape(1, 1), (1, 8)).astype(_f32)

    xpad = jnp.pad(x, ((0, NPAD - N), (0, 0)))
    src4 = jnp.pad(edge_index[0].astype(jnp.int32),
                   (0, EPAD - E)).reshape(NW * NBLK, NB, K)
    dst4 = jnp.pad(edge_index[1].astype(jnp.int32), (0, EPAD - E),
                   constant_values=GARBAGE).reshape(NW * NBLK, NB, K)

    srctab1, a1d, init1 = _ph_a(xpad, w1p, ms16, md16)
    acc1 = _sc_edge(64, 80)(srctab1, a1d, src4, dst4, init1,
                            jnp.zeros((NPAD, 80), _f32))

    srctab2, a2d, init2 = _ph_c(acc1, b1p2d, w2p, a2s16, a2d16)
    acc2 = _sc_edge(128, 144)(srctab2, a2d, src4, dst4, init2,
                              jnp.zeros((NPAD, 144), _f32))

    y8 = _ph_e(acc2, b2r, w8, lb8)
    return y8[:N, :1]


# speedup vs baseline: 1.1149x; 1.1149x over previous
import functools

import numpy as np
import jax
import jax.numpy as jnp
from jax import lax
from jax.experimental import pallas as pl
from jax.experimental.pallas import tpu as pltpu
from jax.experimental.pallas import tpu_sc as plsc

_f32 = jnp.float32

N = 10000
NPAD = 10240
E = 320000
NW = 32
K = 56
NB = 18
NBLK = 10
B0 = 12
B1 = 8
NCH = NB * NBLK
EPW = K * NCH
EPAD = NW * EPW
NTILES = 16
RPT = NPAD // NTILES
GARBAGE = N

_IH16 = np.zeros((64, 16), np.float32)
for _l in range(64):
    _IH16[_l, _l % 8] = 1.0
    _IH16[_l, 8 + _l % 8] = 1.0


def _leaky_exp(t):
    return jnp.exp(jnp.where(t > 0, t, 0.2 * t))


def _ph_a_body(x_ref, w1p, ms, md, st_o, ad_o, init_o):
    xpc = jnp.dot(x_ref[...], w1p[...], preferred_element_type=_f32)
    as16 = jnp.dot(xpc, ms[...], preferred_element_type=_f32)
    ad16 = jnp.dot(xpc, md[...], preferred_element_type=_f32)
    ex = _leaky_exp(as16 + ad16)
    st_o[...] = jnp.concatenate([xpc, as16], axis=1)
    ad_o[...] = ad16
    init_o[...] = jnp.concatenate(
        [xpc * jnp.concatenate([ex[:, 0:8]] * 8, axis=1), ex], axis=1)


_RB = 2048


def _ph_a(xpad, w1p, ms16, md16):
    g = NPAD // _RB
    full = lambda s: pl.BlockSpec(s, lambda i: (0, 0))
    return pl.pallas_call(
        _ph_a_body,
        grid=(g,),
        in_specs=[
            pl.BlockSpec((_RB, 128), lambda i: (i, 0)),
            full((128, 64)), full((64, 16)), full((64, 16)),
        ],
        out_specs=[
            pl.BlockSpec((_RB, 80), lambda i: (i, 0)),
            pl.BlockSpec((_RB, 16), lambda i: (i, 0)),
            pl.BlockSpec((_RB, 80), lambda i: (i, 0)),
        ],
        out_shape=[
            jax.ShapeDtypeStruct((NPAD, 80), _f32),
            jax.ShapeDtypeStruct((NPAD, 16), _f32),
            jax.ShapeDtypeStruct((NPAD, 80), _f32),
        ],
    )(xpad, w1p, ms16, md16)


def _ph_c_body(s0, s1, b1p, w2p, a2s, a2d, st_o, ad_o, init_o):
    sm = s0[...] + s1[...]
    num = sm[:, 0:64]
    den = jnp.concatenate([sm[:, 64:72]] * 8, axis=1)
    o1 = num / (den + 1e-16) + b1p[...]
    h1 = jnp.where(o1 > 0, o1, jnp.exp(jnp.minimum(o1, 0.0)) - 1.0)
    xp2 = jnp.dot(h1, w2p[...], preferred_element_type=_f32)
    as16 = jnp.dot(xp2, a2s[...], preferred_element_type=_f32)
    ad16 = jnp.dot(xp2, a2d[...], preferred_element_type=_f32)
    ex2 = _leaky_exp(as16 + ad16)
    st_o[...] = jnp.concatenate([xp2, as16], axis=1)
    ad_o[...] = ad16
    init_o[...] = jnp.concatenate(
        [xp2 * jnp.concatenate([ex2] * 8, axis=1), ex2], axis=1)


def _ph_c(acc1, b1p2d, w2p, a2s16, a2d16):
    g = NPAD // _RB
    full = lambda s: pl.BlockSpec(s, lambda i: (0, 0))
    return pl.pallas_call(
        _ph_c_body,
        grid=(g,),
        in_specs=[
            pl.BlockSpec((_RB, 80), lambda i: (i, 0)),
            pl.BlockSpec((_RB, 80), lambda i: (i + NPAD // _RB, 0)),
            full((1, 64)),
            full((64, 128)), full((128, 16)), full((128, 16)),
        ],
        out_specs=[
            pl.BlockSpec((_RB, 144), lambda i: (i, 0)),
            pl.BlockSpec((_RB, 16), lambda i: (i, 0)),
            pl.BlockSpec((_RB, 144), lambda i: (i, 0)),
        ],
        out_shape=[
            jax.ShapeDtypeStruct((NPAD, 144), _f32),
            jax.ShapeDtypeStruct((NPAD, 16), _f32),
            jax.ShapeDtypeStruct((NPAD, 144), _f32),
        ],
    )(acc1, acc1, b1p2d, w2p, a2s16, a2d16)


def _ph_e_body(s0, s1, b2r, w8, lb8, y_o):
    sm = s0[...] + s1[...]
    num = sm[:, 0:128]
    den = jnp.concatenate([sm[:, 128:144]] * 8, axis=1)
    o2 = num / (den + 1e-16) + b2r[...]
    y_o[...] = jnp.dot(o2, w8[...], preferred_element_type=_f32) + lb8[...]


def _ph_e(acc2, b2r, w8, lb8):
    g = NPAD // _RB
    full = lambda s: pl.BlockSpec(s, lambda i: (0, 0))
    return pl.pallas_call(
        _ph_e_body,
        grid=(g,),
        in_specs=[
            pl.BlockSpec((_RB, 144), lambda i: (i, 0)),
            pl.BlockSpec((_RB, 144), lambda i: (i + NPAD // _RB, 0)),
            full((1, 128)), full((128, 8)), full((1, 8)),
        ],
        out_specs=pl.BlockSpec((_RB, 8), lambda i: (i, 0)),
        out_shape=jax.ShapeDtypeStruct((NPAD, 8), _f32),
    )(acc2, acc2, b2r, w8, lb8)


def _make_sc_edge(D, DO):
    mesh = plsc.VectorSubcoreMesh(core_axis_name="c", subcore_axis_name="s")
    vm = pltpu.VMEM
    sem = pltpu.SemaphoreType.DMA

    @functools.partial(
        pl.kernel,
        mesh=mesh,
        out_type=jax.ShapeDtypeStruct((2 * NPAD, DO), _f32),
        compiler_params=pltpu.CompilerParams(use_tc_tiling_on_sc=False),
        scratch_types=[
            [vm((NB, K), jnp.int32)] * 2, [vm((NB, K), jnp.int32)] * 2,
            [vm((K, 16), _f32)] * 2, [vm((K, DO), _f32)] * 2,
            [vm((K, DO), _f32)] * 2,
            pltpu.VMEM_SHARED((NPAD, DO), _f32),
            [sem] * 2, [sem] * 2, [sem] * 2, [sem] * 2, [sem] * 2,
        ],
    )
    def sc(st_h, ad_h, src4_h, dst4_h, init_h, zro_h, out_h,
           ibs, ibd, bv, fv, ov, snum, sis, sid, ga, gb, gs):
        c = lax.axis_index("c")
        s = lax.axis_index("s")
        row0 = s * RPT

        @pl.when(c == 0)
        def _():
            pltpu.sync_copy(init_h.at[pl.ds(row0, RPT)],
                            snum.at[pl.ds(row0, RPT)])

        @pl.when(c == 1)
        def _():
            pltpu.sync_copy(zro_h.at[pl.ds(row0, RPT)],
                            snum.at[pl.ds(row0, RPT)])

        plsc.subcore_barrier()
        bbase = c * (NTILES * B0) + s * (B0 + (B1 - B0) * c)
        nblk = B0 + (B1 - B0) * c

        def idx_start(b, q):
            pltpu.make_async_copy(src4_h.at[bbase + b], ibs[q], sis[q]).start()
            pltpu.make_async_copy(dst4_h.at[bbase + b], ibd[q], sid[q]).start()

        def idx_wait(q):
            pltpu.make_async_copy(src4_h.at[bbase], ibs[q], sis[q]).wait()
            pltpu.make_async_copy(dst4_h.at[bbase], ibd[q], sid[q]).wait()

        def g_start(q, j, p):
            pltpu.make_async_copy(st_h.at[ibs[q].at[j]], fv[p], ga[p]).start()
            pltpu.make_async_copy(ad_h.at[ibd[q].at[j]], bv[p], gb[p]).start()

        def g_wait(q, p):
            pltpu.make_async_copy(st_h.at[ibs[q].at[0]], fv[p], ga[p]).wait()
            pltpu.make_async_copy(ad_h.at[ibd[q].at[0]], bv[p], gb[p]).wait()

        def sc_start(q, j, p):
            pltpu.make_async_copy(ov[p], snum.at[ibd[q].at[j]],
                                  gs[p]).start(add=True)

        def sc_drain(q, p):
            pltpu.make_async_copy(ov[p], snum.at[ibd[q].at[0]], gs[p]).wait()

        idx_start(0, 0)

        def blockpair(b2, cr):
            for q in (0, 1):
                b = b2 * 2 + q
                idx_wait(q)
                g_start(q, 0, 0)
                g_start(q, 1, 1)

                def pairbody(pj, cr2, q=q, b=b):
                    for p in (0, 1):
                        j = pj * 2 + p
                        ch = b * NB + j
                        g_wait(q, p)

                        @pl.when(ch >= 2)
                        def _(q=q, p=p):
                            sc_drain(q, p)

                        def ebody(i2, cr3, q=q, p=p):
                            ee = (i2 * 2, i2 * 2 + 1)
                            exs = [_leaky_exp(
                                fv[p][e, D:D + 16] + bv[p][e, :]) for e in ee]
                            for e, ex in zip(ee, exs):
                                fs = [fv[p][e, k * 16:(k + 1) * 16]
                                      for k in range(D // 16)]
                                outs = [f * ex for f in fs]
                                for k in range(D // 16):
                                    ov[p][e, k * 16:(k + 1) * 16] = outs[k]
                                ov[p][e, D:D + 16] = ex
                            return cr3

                        lax.fori_loop(0, K // 2, ebody, 0, unroll=2)
                        sc_start(q, j, p)

                        @pl.when(j + 2 < NB)
                        def _(q=q, j=j, p=p):
                            g_start(q, j + 2, p)

                    @pl.when((pj == 0) & (b + 1 < nblk))
                    def _(q=q, b=b):
                        idx_start(b + 1, 1 - q)
                    return cr2

                lax.fori_loop(0, NB // 2, pairbody, 0)
            return cr

        lax.fori_loop(0, nblk // 2, blockpair, 0)
        sc_drain(1, 0)
        sc_drain(1, 1)
        plsc.subcore_barrier()
        pltpu.sync_copy(snum.at[pl.ds(row0, RPT)],
                        out_h.at[pl.ds(c * NPAD + row0, RPT)])

    return sc


_sc_cache = {}


def _sc_edge(D, DO):
    if (D, DO) not in _sc_cache:
        _sc_cache[(D, DO)] = _make_sc_edge(D, DO)
    return _sc_cache[(D, DO)]


def kernel(x, edge_index, W1, a1_src, a1_dst, b1, W2, a2_src, a2_dst, b2,
           lin_W, lin_b):
    w1p = W1.reshape(128, 8, 8).transpose(0, 2, 1).reshape(128, 64)
    ih16 = jnp.asarray(_IH16)
    ms16 = a1_src.T.reshape(64, 1) * ih16
    md16 = a1_dst.T.reshape(64, 1) * ih16
    b1p2d = b1.reshape(8, 8).T.reshape(1, 64)
    w2p = W2.reshape(8, 8, 128).transpose(1, 0, 2).reshape(64, 128)
    a2s16 = jnp.tile(a2_src.reshape(128, 1), (1, 16))
    a2d16 = jnp.tile(a2_dst.reshape(128, 1), (1, 16))
    b2r = b2.reshape(1, 128)
    w8 = jnp.tile(lin_W.reshape(128, 1), (1, 8))
    lb8 = jnp.broadcast_to(lin_b.reshape(1, 1), (1, 8)).astype(_f32)

    xpad = jnp.pad(x, ((0, NPAD - N), (0, 0)))
    src4 = jnp.pad(edge_index[0].astype(jnp.int32),
                   (0, EPAD - E)).reshape(NW * NBLK, NB, K)
    dst4 = jnp.pad(edge_index[1].astype(jnp.int32), (0, EPAD - E),
                   constant_values=GARBAGE).reshape(NW * NBLK, NB, K)

    srctab1, a1d, init1 = _ph_a(xpad, w1p, ms16, md16)
    acc1 = _sc_edge(64, 80)(srctab1, a1d, src4, dst4, init1,
                            jnp.zeros((NPAD, 80), _f32))

    srctab2, a2d, init2 = _ph_c(acc1, b1p2d, w2p, a2s16, a2d16)
    acc2 = _sc_edge(128, 144)(srctab2, a2d, src4, dst4, init2,
                              jnp.zeros((NPAD, 144), _f32))

    y8 = _ph_e(acc2, b2r, w8, lb8)
    return y8[:N, :1]
